# baseline (speedup 1.0000x reference)
import functools

import jax
import jax.numpy as jnp
from jax import lax
from jax.experimental import pallas as pl
from jax.experimental.pallas import tpu as pltpu
from jax.experimental.pallas import tpu_sc as plsc

NN = 10000
NP = 10240
NA = 10240
EE = 160000
EP = 163840
NSUB = 16
WIN = 128
NWIN = EP // (NSUB * WIN)
GG = 64
HH = 512
NB = 512
RPS = NA // NSUB


def _pipelined_accumulate(load, out_view, acc, z128, dst_row, dbuf,
                          sid, nwin):
    r0 = sid * RPS
    pltpu.sync_copy(z128.at[pl.ds(r0, RPS)], acc.at[pl.ds(r0, RPS)])
    plsc.subcore_barrier()

    @pl.loop(0, nwin)
    def _(w):
        load(w)
        pltpu.sync_copy(dbuf, acc.at[dst_row(w)], add=True)

    plsc.subcore_barrier()
    pltpu.sync_copy(acc.at[pl.ds(r0, RPS)], out_view.at[pl.ds(r0, RPS)])
    plsc.subcore_barrier()


def _seg_chunk(h_view, s_view, acc, z128, src_buf, dst_buf, dbuf, sid):
    def load(w):
        pltpu.sync_copy(h_view.at[src_buf.at[w]], dbuf)

    _pipelined_accumulate(load, s_view, acc, z128,
                          lambda w: dst_buf.at[w], dbuf, sid, NWIN)


_SC_SCRATCH = [
    pltpu.VMEM((NWIN, WIN), jnp.int32),
    pltpu.VMEM((NWIN, WIN), jnp.int32),
    pltpu.VMEM((WIN, 128), jnp.float32),
    pltpu.VMEM_SHARED((NA, 128), jnp.float32),
]


def _sc_ea(ea128, dstR, z128):
    mesh = plsc.VectorSubcoreMesh(core_axis_name="c", subcore_axis_name="s")
    nw2 = NWIN // 2

    @functools.partial(
        pl.kernel, mesh=mesh,
        out_type=jax.ShapeDtypeStruct((2, NP, 128), jnp.float32),
        scratch_types=list(_SC_SCRATCH),
    )
    def k(ea_hbm, dst_hbm, z128_hbm, eax_hbm,
          src_buf, dst_buf, dbuf, acc):
        core = lax.axis_index("c")
        sid = lax.axis_index("s")
        pltpu.sync_copy(dst_hbm.at[sid], dst_buf)
        w0 = core * nw2

        def load(w):
            pltpu.sync_copy(
                ea_hbm.at[pl.ds(sid * (NWIN * WIN) + (w0 + w) * WIN, WIN)],
                dbuf)

        _pipelined_accumulate(load, eax_hbm.at[core], acc, z128_hbm,
                              lambda w: dst_buf.at[w0 + w], dbuf, sid, nw2)

    return k(ea128, dstR, z128)


def _sc_layer0(x2, srcR, dstR, z128):
    mesh = plsc.VectorSubcoreMesh(core_axis_name="c", subcore_axis_name="s")

    @functools.partial(
        pl.kernel, mesh=mesh,
        out_type=jax.ShapeDtypeStruct((2, NP, 128), jnp.float32),
        scratch_types=list(_SC_SCRATCH),
    )
    def k(x_hbm, src_hbm, dst_hbm, z128_hbm, s_hbm,
          src_buf, dst_buf, dbuf, acc):
        core = lax.axis_index("c")
        sid = lax.axis_index("s")
        pltpu.sync_copy(src_hbm.at[sid], src_buf)
        pltpu.sync_copy(dst_hbm.at[sid], dst_buf)
        _seg_chunk(x_hbm.at[core], s_hbm.at[core], acc, z128_hbm,
                   src_buf, dst_buf, dbuf, sid)

    return k(x2, srcR, dstR, z128)


def _sc_layer(h4, srcR, dstR, z128):
    mesh = plsc.VectorSubcoreMesh(core_axis_name="c", subcore_axis_name="s")

    @functools.partial(
        pl.kernel, mesh=mesh,
        out_type=jax.ShapeDtypeStruct((4, NP, 128), jnp.float32),
        scratch_types=list(_SC_SCRATCH),
    )
    def k(h_hbm, src_hbm, dst_hbm, z128_hbm, s_hbm,
          src_buf, dst_buf, dbuf, acc):
        core = lax.axis_index("c")
        sid = lax.axis_index("s")
        pltpu.sync_copy(src_hbm.at[sid], src_buf)
        pltpu.sync_copy(dst_hbm.at[sid], dst_buf)

        @pl.loop(0, 2)
        def _(j):
            c = core * 2 + j
            _seg_chunk(h_hbm.at[c], s_hbm.at[c], acc, z128_hbm,
                       src_buf, dst_buf, dbuf, sid)

    return k(h4, srcR, dstR, z128)


def _hidden_block(h_ref, s_ref, e_ref, wi_ref, wj_ref, we_ref, b_ref, nc):
    e_blk = e_ref[0] + e_ref[1]
    deg = e_blk[:, 16:17]
    acc = deg * b_ref[...]
    acc += jnp.dot(e_blk, we_ref[...], preferred_element_type=jnp.float32)
    for c in range(nc):
        hc = h_ref[c] * deg
        acc += jnp.dot(hc, wi_ref[pl.ds(c * 128, 128), :],
                       preferred_element_type=jnp.float32)
        acc += jnp.dot(s_ref[c], wj_ref[pl.ds(c * 128, 128), :],
                       preferred_element_type=jnp.float32)
    return jnp.maximum(acc, 0.0)


def _tc_layer(hch, sch, eax, wit, wjt, wept, bm):
    nc = hch.shape[0]

    def body(h_ref, s_ref, e_ref, wi_ref, wj_ref, we_ref, b_ref, o_ref):
        hnew = _hidden_block(h_ref, s_ref, e_ref, wi_ref, wj_ref, we_ref,
                             b_ref, nc)
        for c in range(4):
            o_ref[c] = hnew[:, c * 128:(c + 1) * 128]

    return pl.pallas_call(
        body,
        grid=(NP // NB,),
        in_specs=[
            pl.BlockSpec((nc, NB, 128), lambda i: (0, i, 0)),
            pl.BlockSpec((nc, NB, 128), lambda i: (0, i, 0)),
            pl.BlockSpec((2, NB, 128), lambda i: (0, i, 0)),
            pl.BlockSpec((nc * 128, HH), lambda i: (0, 0)),
            pl.BlockSpec((nc * 128, HH), lambda i: (0, 0)),
            pl.BlockSpec((128, HH), lambda i: (0, 0)),
            pl.BlockSpec((1, HH), lambda i: (0, 0)),
        ],
        out_specs=pl.BlockSpec((4, NB, 128), lambda i: (0, i, 0)),
        out_shape=jax.ShapeDtypeStruct((4, NP, 128), jnp.float32),
    )(hch, sch, eax, wit, wjt, wept, bm)


def _tc_final(hch, sch, eax, wit, wjt, wept, bm, batch_p, wlpt, blp):
    def body(h_ref, s_ref, e_ref, wi_ref, wj_ref, we_ref, b_ref,
             bat_ref, wl_ref, bl_ref, o_ref, pool, cnt):
        i = pl.program_id(0)

        @pl.when(i == 0)
        def _():
            pool[...] = jnp.zeros_like(pool)
            cnt[...] = jnp.zeros_like(cnt)

        hnew = _hidden_block(h_ref, s_ref, e_ref, wi_ref, wj_ref, we_ref,
                             b_ref, 4)
        rid = i * NB + lax.broadcasted_iota(jnp.int32, (NB, 1), 0)
        hnew = jnp.where(rid < NN, hnew, 0.0)
        bat = bat_ref[...][:, 0]
        gids = lax.broadcasted_iota(jnp.int32, (GG, NB), 0)
        oh = jnp.where(gids == bat[None, :], 1.0, 0.0)
        pool[...] += jnp.dot(oh, hnew, preferred_element_type=jnp.float32)
        cnt[...] += jnp.broadcast_to(
            jnp.sum(oh, axis=1, keepdims=True), (GG, 128))

        @pl.when(i == NP // NB - 1)
        def _():
            pooled = pool[...] / jnp.maximum(cnt[...][:, :1], 1.0)
            logits = jnp.dot(pooled, wl_ref[...],
                             preferred_element_type=jnp.float32) + bl_ref[...]
            m = jnp.max(logits, axis=1, keepdims=True)
            lse = jnp.log(jnp.sum(jnp.exp(logits - m), axis=1,
                                  keepdims=True)) + m
            o_ref[...] = logits - lse

    return pl.pallas_call(
        body,
        grid=(NP // NB,),
        in_specs=[
            pl.BlockSpec((4, NB, 128), lambda i: (0, i, 0)),
            pl.BlockSpec((4, NB, 128), lambda i: (0, i, 0)),
            pl.BlockSpec((2, NB, 128), lambda i: (0, i, 0)),
            pl.BlockSpec((HH, HH), lambda i: (0, 0)),
            pl.BlockSpec((HH, HH), lambda i: (0, 0)),
            pl.BlockSpec((128, HH), lambda i: (0, 0)),
            pl.BlockSpec((1, HH), lambda i: (0, 0)),
            pl.BlockSpec((NB, 1), lambda i: (i, 0)),
            pl.BlockSpec((HH, 128), lambda i: (0, 0)),
            pl.BlockSpec((1, 128), lambda i: (0, 0)),
        ],
        out_specs=pl.BlockSpec((GG, 128), lambda i: (0, 0)),
        out_shape=jax.ShapeDtypeStruct((GG, 128), jnp.float32),
        scratch_shapes=[
            pltpu.VMEM((GG, HH), jnp.float32),
            pltpu.VMEM((GG, 128), jnp.float32),
        ],
    )(hch, sch, eax, wit, wjt, wept, bm, batch_p, wlpt, blp)


def kernel(x, edge_index, edge_attr, batch, W0, b0, W1, b1, W2, b2, Wl, bl):
    f32 = jnp.float32
    src = edge_index[0]
    dst = edge_index[1]

    pad = EP - EE
    pad_ids = jnp.arange(pad, dtype=jnp.int32) % 16
    src_p = jnp.concatenate([src, pad_ids])
    dst_p = jnp.concatenate([dst, NN + pad_ids])
    srcR = src_p.reshape(NSUB, NWIN, WIN)
    dstR = dst_p.reshape(NSUB, NWIN, WIN)

    ea128 = jnp.zeros((EP, 128), f32)
    ea128 = ea128.at[:EE, :16].set(edge_attr)
    ea128 = ea128.at[:EE, 16].set(1.0)

    x_p = jnp.zeros((NP, 256), f32).at[:NN].set(x)
    x2 = x_p.reshape(NP, 2, 128).transpose(1, 0, 2)
    z128 = jnp.zeros((NP, 128), f32)
    batch_p = jnp.full((NP, 1), 1 << 30, jnp.int32).at[:NN, 0].set(batch)

    def parts(W, d):
        wit = W[:, :d].T
        wjt = W[:, d:2 * d].T
        wept = jnp.zeros((128, HH), f32).at[:16].set(W[:, 2 * d:].T)
        return wit, wjt, wept

    wi0, wj0, we0 = parts(W0, 256)
    wi1, wj1, we1 = parts(W1, 512)
    wi2, wj2, we2 = parts(W2, 512)
    wlpt = jnp.zeros((HH, 128), f32).at[:, :4].set(Wl.T)
    blp = jnp.full((1, 128), -1e30, f32).at[0, :4].set(bl)

    eax = _sc_ea(ea128, dstR, z128)
    s1 = _sc_layer0(x2, srcR, dstR, z128)
    h1 = _tc_layer(x2, s1, eax, wi0, wj0, we0, b0.reshape(1, HH))
    s2 = _sc_layer(h1, srcR, dstR, z128)
    h2 = _tc_layer(h1, s2, eax, wi1, wj1, we1, b1.reshape(1, HH))
    s3 = _sc_layer(h2, srcR, dstR, z128)
    out = _tc_final(h2, s3, eax, wi2, wj2, we2, b2.reshape(1, HH),
                    batch_p, wlpt, blp)
    return out[:, :4]


# speedup vs baseline: 1.1964x; 1.1964x over previous
import functools

import jax
import jax.numpy as jnp
from jax import lax
from jax.experimental import pallas as pl
from jax.experimental.pallas import tpu as pltpu
from jax.experimental.pallas import tpu_sc as plsc

NN = 10000
NP = 10240
NA = 10240
EE = 160000
EP = 163840
NSUB = 16
WIN = 128
NWIN = EP // (NSUB * WIN)
GG = 64
HH = 512
NB = 512
RPS = NA // NSUB


HW = NWIN // 2


def _pipelined_accumulate(load, load_idx, out_view, acc, z128, dst_buf,
                          dbuf, ssem, sid, nhalves):
    r0 = sid * RPS
    pltpu.sync_copy(z128.at[pl.ds(r0, RPS)], acc.at[pl.ds(r0, RPS)])
    plsc.subcore_barrier()

    def make_scat(wl, b):
        return pltpu.make_async_copy(dbuf.at[b], acc.at[dst_buf.at[wl]],
                                     ssem.at[b])

    for half in range(nhalves):
        load_idx(half)

        @pl.loop(0, HW)
        def _(wl):
            b = lax.rem(wl, 2)

            @pl.when(wl >= 2)
            def _():
                make_scat(wl - 2, b).wait()

            load(half * HW + wl, wl, b)
            make_scat(wl, b).start(add=True)

        @pl.loop(0, 2)
        def _(k):
            wl = HW - 2 + k
            make_scat(wl, lax.rem(wl, 2)).wait()

    plsc.subcore_barrier()
    pltpu.sync_copy(acc.at[pl.ds(r0, RPS)], out_view.at[pl.ds(r0, RPS)])
    plsc.subcore_barrier()


def _seg_chunk(h_view, s_view, acc, z128, src_hbm, dst_hbm, src_buf,
               dst_buf, dbuf, ssem, sid):
    def load_idx(half):
        pltpu.sync_copy(src_hbm.at[sid, pl.ds(half * HW, HW)], src_buf)
        pltpu.sync_copy(dst_hbm.at[sid, pl.ds(half * HW, HW)], dst_buf)

    def load(w, wl, b):
        pltpu.sync_copy(h_view.at[src_buf.at[wl]], dbuf.at[b])

    _pipelined_accumulate(load, load_idx, s_view, acc, z128, dst_buf,
                          dbuf, ssem, sid, 2)


_SC_SCRATCH = [
    pltpu.VMEM((HW, WIN), jnp.int32),
    pltpu.VMEM((HW, WIN), jnp.int32),
    pltpu.VMEM((2, WIN, 128), jnp.float32),
    pltpu.VMEM_SHARED((NA, 128), jnp.float32),
    pltpu.SemaphoreType.DMA((2,)),
]


def _sc_ea(ea128, dstR, z128):
    mesh = plsc.VectorSubcoreMesh(core_axis_name="c", subcore_axis_name="s")

    @functools.partial(
        pl.kernel, mesh=mesh,
        out_type=jax.ShapeDtypeStruct((2, NP, 128), jnp.float32),
        scratch_types=list(_SC_SCRATCH),
    )
    def k(ea_hbm, dst_hbm, z128_hbm, eax_hbm,
          src_buf, dst_buf, dbuf, acc, ssem):
        core = lax.axis_index("c")
        sid = lax.axis_index("s")

        def load_idx(half):
            pltpu.sync_copy(dst_hbm.at[sid, pl.ds(core * HW, HW)], dst_buf)

        def load(w, wl, b):
            pltpu.sync_copy(
                ea_hbm.at[pl.ds(sid * (NWIN * WIN) + (core * HW + wl) * WIN,
                                WIN)],
                dbuf.at[b])

        _pipelined_accumulate(load, load_idx, eax_hbm.at[core], acc,
                              z128_hbm, dst_buf, dbuf, ssem, sid, 1)

    return k(ea128, dstR, z128)


def _sc_layer0(x2, srcR, dstR, z128):
    mesh = plsc.VectorSubcoreMesh(core_axis_name="c", subcore_axis_name="s")

    @functools.partial(
        pl.kernel, mesh=mesh,
        out_type=jax.ShapeDtypeStruct((2, NP, 128), jnp.float32),
        scratch_types=list(_SC_SCRATCH),
    )
    def k(x_hbm, src_hbm, dst_hbm, z128_hbm, s_hbm,
          src_buf, dst_buf, dbuf, acc, ssem):
        core = lax.axis_index("c")
        sid = lax.axis_index("s")
        _seg_chunk(x_hbm.at[core], s_hbm.at[core], acc, z128_hbm,
                   src_hbm, dst_hbm, src_buf, dst_buf, dbuf, ssem, sid)

    return k(x2, srcR, dstR, z128)


def _sc_layer(h4, srcR, dstR, z128):
    mesh = plsc.VectorSubcoreMesh(core_axis_name="c", subcore_axis_name="s")

    @functools.partial(
        pl.kernel, mesh=mesh,
        out_type=jax.ShapeDtypeStruct((4, NP, 128), jnp.float32),
        scratch_types=list(_SC_SCRATCH),
    )
    def k(h_hbm, src_hbm, dst_hbm, z128_hbm, s_hbm,
          src_buf, dst_buf, dbuf, acc, ssem):
        core = lax.axis_index("c")
        sid = lax.axis_index("s")

        @pl.loop(0, 2)
        def _(j):
            c = core * 2 + j
            _seg_chunk(h_hbm.at[c], s_hbm.at[c], acc, z128_hbm,
                       src_hbm, dst_hbm, src_buf, dst_buf, dbuf, ssem, sid)

    return k(h4, srcR, dstR, z128)


def _hidden_block(h_ref, s_ref, e_ref, wi_ref, wj_ref, we_ref, b_ref, nc):
    e_blk = e_ref[0] + e_ref[1]
    deg = e_blk[:, 16:17]
    acc = deg * b_ref[...]
    acc += jnp.dot(e_blk, we_ref[...], preferred_element_type=jnp.float32)
    for c in range(nc):
        hc = h_ref[c] * deg
        acc += jnp.dot(hc, wi_ref[pl.ds(c * 128, 128), :],
                       preferred_element_type=jnp.float32)
        acc += jnp.dot(s_ref[c], wj_ref[pl.ds(c * 128, 128), :],
                       preferred_element_type=jnp.float32)
    return jnp.maximum(acc, 0.0)


def _tc_layer(hch, sch, eax, wit, wjt, wept, bm):
    nc = hch.shape[0]

    def body(h_ref, s_ref, e_ref, wi_ref, wj_ref, we_ref, b_ref, o_ref):
        hnew = _hidden_block(h_ref, s_ref, e_ref, wi_ref, wj_ref, we_ref,
                             b_ref, nc)
        for c in range(4):
            o_ref[c] = hnew[:, c * 128:(c + 1) * 128]

    return pl.pallas_call(
        body,
        grid=(NP // NB,),
        in_specs=[
            pl.BlockSpec((nc, NB, 128), lambda i: (0, i, 0)),
            pl.BlockSpec((nc, NB, 128), lambda i: (0, i, 0)),
            pl.BlockSpec((2, NB, 128), lambda i: (0, i, 0)),
            pl.BlockSpec((nc * 128, HH), lambda i: (0, 0)),
            pl.BlockSpec((nc * 128, HH), lambda i: (0, 0)),
            pl.BlockSpec((128, HH), lambda i: (0, 0)),
            pl.BlockSpec((1, HH), lambda i: (0, 0)),
        ],
        out_specs=pl.BlockSpec((4, NB, 128), lambda i: (0, i, 0)),
        out_shape=jax.ShapeDtypeStruct((4, NP, 128), jnp.float32),
    )(hch, sch, eax, wit, wjt, wept, bm)


def _tc_final(hch, sch, eax, wit, wjt, wept, bm, batch_p, wlpt, blp):
    def body(h_ref, s_ref, e_ref, wi_ref, wj_ref, we_ref, b_ref,
             bat_ref, wl_ref, bl_ref, o_ref, pool, cnt):
        i = pl.program_id(0)

        @pl.when(i == 0)
        def _():
            pool[...] = jnp.zeros_like(pool)
            cnt[...] = jnp.zeros_like(cnt)

        hnew = _hidden_block(h_ref, s_ref, e_ref, wi_ref, wj_ref, we_ref,
                             b_ref, 4)
        rid = i * NB + lax.broadcasted_iota(jnp.int32, (NB, 1), 0)
        hnew = jnp.where(rid < NN, hnew, 0.0)
        bat = bat_ref[...][:, 0]
        gids = lax.broadcasted_iota(jnp.int32, (GG, NB), 0)
        oh = jnp.where(gids == bat[None, :], 1.0, 0.0)
        pool[...] += jnp.dot(oh, hnew, preferred_element_type=jnp.float32)
        cnt[...] += jnp.broadcast_to(
            jnp.sum(oh, axis=1, keepdims=True), (GG, 128))

        @pl.when(i == NP // NB - 1)
        def _():
            pooled = pool[...] / jnp.maximum(cnt[...][:, :1], 1.0)
            logits = jnp.dot(pooled, wl_ref[...],
                             preferred_element_type=jnp.float32) + bl_ref[...]
            m = jnp.max(logits, axis=1, keepdims=True)
            lse = jnp.log(jnp.sum(jnp.exp(logits - m), axis=1,
                                  keepdims=True)) + m
            o_ref[...] = logits - lse

    return pl.pallas_call(
        body,
        grid=(NP // NB,),
        in_specs=[
            pl.BlockSpec((4, NB, 128), lambda i: (0, i, 0)),
            pl.BlockSpec((4, NB, 128), lambda i: (0, i, 0)),
            pl.BlockSpec((2, NB, 128), lambda i: (0, i, 0)),
            pl.BlockSpec((HH, HH), lambda i: (0, 0)),
            pl.BlockSpec((HH, HH), lambda i: (0, 0)),
            pl.BlockSpec((128, HH), lambda i: (0, 0)),
            pl.BlockSpec((1, HH), lambda i: (0, 0)),
            pl.BlockSpec((NB, 1), lambda i: (i, 0)),
            pl.BlockSpec((HH, 128), lambda i: (0, 0)),
            pl.BlockSpec((1, 128), lambda i: (0, 0)),
        ],
        out_specs=pl.BlockSpec((GG, 128), lambda i: (0, 0)),
        out_shape=jax.ShapeDtypeStruct((GG, 128), jnp.float32),
        scratch_shapes=[
            pltpu.VMEM((GG, HH), jnp.float32),
            pltpu.VMEM((GG, 128), jnp.float32),
        ],
    )(hch, sch, eax, wit, wjt, wept, bm, batch_p, wlpt, blp)


def kernel(x, edge_index, edge_attr, batch, W0, b0, W1, b1, W2, b2, Wl, bl):
    f32 = jnp.float32
    src = edge_index[0]
    dst = edge_index[1]

    pad = EP - EE
    pad_ids = jnp.arange(pad, dtype=jnp.int32) % 16
    src_p = jnp.concatenate([src, pad_ids])
    dst_p = jnp.concatenate([dst, NN + pad_ids])
    srcR = src_p.reshape(NSUB, NWIN, WIN)
    dstR = dst_p.reshape(NSUB, NWIN, WIN)

    ea128 = jnp.zeros((EP, 128), f32)
    ea128 = ea128.at[:EE, :16].set(edge_attr)
    ea128 = ea128.at[:EE, 16].set(1.0)

    x_p = jnp.zeros((NP, 256), f32).at[:NN].set(x)
    x2 = x_p.reshape(NP, 2, 128).transpose(1, 0, 2)
    z128 = jnp.zeros((NP, 128), f32)
    batch_p = jnp.full((NP, 1), 1 << 30, jnp.int32).at[:NN, 0].set(batch)

    def parts(W, d):
        wit = W[:, :d].T
        wjt = W[:, d:2 * d].T
        wept = jnp.zeros((128, HH), f32).at[:16].set(W[:, 2 * d:].T)
        return wit, wjt, wept

    wi0, wj0, we0 = parts(W0, 256)
    wi1, wj1, we1 = parts(W1, 512)
    wi2, wj2, we2 = parts(W2, 512)
    wlpt = jnp.zeros((HH, 128), f32).at[:, :4].set(Wl.T)
    blp = jnp.full((1, 128), -1e30, f32).at[0, :4].set(bl)

    eax = _sc_ea(ea128, dstR, z128)
    s1 = _sc_layer0(x2, srcR, dstR, z128)
    h1 = _tc_layer(x2, s1, eax, wi0, wj0, we0, b0.reshape(1, HH))
    s2 = _sc_layer(h1, srcR, dstR, z128)
    h2 = _tc_layer(h1, s2, eax, wi1, wj1, we1, b1.reshape(1, HH))
    s3 = _sc_layer(h2, srcR, dstR, z128)
    out = _tc_final(h2, s3, eax, wi2, wj2, we2, b2.reshape(1, HH),
                    batch_p, wlpt, blp)
    return out[:, :4]
